# Initial kernel scaffold; baseline (speedup 1.0000x reference)
import jax
import jax.numpy as jnp
from jax.experimental import pallas as pl


def kernel(x, x0, weights, edge_index, W_gcn, b_gcn, W_sage_l, W_sage_r, b_sage, W_gin, b_gin):
    raise NotImplementedError("write your pallas kernel here")


# speedup vs baseline: 6.9716x; 6.9716x over previous
import functools

import jax
import jax.numpy as jnp
from jax import lax
from jax.experimental import pallas as pl
from jax.experimental.pallas import tpu as pltpu
from jax.experimental.pallas import tpu_sc as plsc

NN = 10000
EE = 320000
DD = 128

NC = 2
NS = 16
CH = 128
EP = 327680
EPT = EP // NS
RPT = EPT // CH
NPAD = 10240
NH = NPAD // 2
NR = NPAD // 16
STR = NPAD // NS
DW = 16

BLK = 1024
NB = NPAD // BLK
HB = NB // 2


def _sc_mesh():
  return plsc.VectorSubcoreMesh(
      core_axis_name="c", subcore_axis_name="s", num_cores=NC, num_subcores=NS)


def _deg_body(e1_hbm, out_hbm, idx_v, acc8, sem):
  c = lax.axis_index("c")
  s = lax.axis_index("s")

  zero16 = jnp.zeros((16,), jnp.float32)

  def _z(i, carry):
    acc8[pl.ds(16 * i, 16)] = zero16
    return carry
  lax.fori_loop(0, NR * DD // 16, _z, 0)

  ones16 = jnp.ones((16,), jnp.float32)
  lanes = lax.iota(jnp.int32, 16)
  lane7 = lanes % 8
  m_lo = lanes < 8
  m_hi = lanes >= 8

  base = (1 - c) * EP + s * EPT

  def _chunk(j, carry):
    pltpu.sync_copy(e1_hbm.at[pl.ds(base + j * CH, CH)], idx_v)

    def _sub(k, c2):
      iv = idx_v[pl.ds(k * 16, 16)]
      flat = jnp.where(iv >= NH, 16 * (iv - NH) + 8, 16 * iv) + lane7
      plsc.addupdate_scatter(acc8, [flat], ones16, mask=m_lo)
      plsc.addupdate_scatter(acc8, [flat], ones16, mask=m_hi)
      return c2
    lax.fori_loop(0, CH // 16, _sub, 0)
    return carry
  lax.fori_loop(0, RPT, _chunk, 0)

  w = (c * NS + s) * NR * DD
  pltpu.sync_copy(acc8, out_hbm.at[pl.ds(w, NR * DD)])


def _degrees(e1):
  f = functools.partial(
      pl.kernel,
      out_type=jax.ShapeDtypeStruct((2 * NS * NR * DD,), jnp.float32),
      mesh=_sc_mesh(),
      compiler_params=pltpu.CompilerParams(needs_layout_passes=False),
      scratch_types=[
          pltpu.VMEM((CH,), jnp.int32),
          pltpu.VMEM((NR * DD,), jnp.float32),
          pltpu.SemaphoreType.DMA,
      ],
  )(_deg_body)
  return f(e1)


def _scale_body(x_ref, part_ref, z_ref, di_ref):
  i = pl.program_id(0)
  xb = x_ref[...]
  t = jnp.sum(part_ref[...], axis=1)
  tt = t.reshape(2, BLK // 8, 8, 16)
  lo = jnp.sum(tt[:, :, :, 0:8], axis=-1).reshape(2, BLK)
  hi = jnp.sum(tt[:, :, :, 8:16], axis=-1).reshape(2, BLK)
  deg = jnp.where(i < HB, lo, hi)
  di = jnp.maximum(deg[0], 1.0)
  do = jnp.maximum(deg[1], 1.0)
  z_ref[0] = xb
  z_ref[1] = xb * lax.rsqrt(do)[:, None]
  di_ref[...] = jnp.broadcast_to(di[:, None], (BLK, 8))


def _planes(x, partials):
  return pl.pallas_call(
      _scale_body,
      grid=(NB,),
      in_specs=[
          pl.BlockSpec((BLK, DD), lambda i: (i, 0)),
          pl.BlockSpec((2, NS, BLK // 8, DD), lambda i: (0, 0, i % HB, 0)),
      ],
      out_specs=[
          pl.BlockSpec((2, BLK, DD), lambda i: (0, i, 0)),
          pl.BlockSpec((BLK, 8), lambda i: (i, 0)),
      ],
      out_shape=[
          jax.ShapeDtypeStruct((2, NPAD, DD), jnp.float32),
          jax.ShapeDtypeStruct((NPAD, 8), jnp.float32),
      ],
  )(x, partials)


def _feat_body(z_hbm, e1_hbm, out_hbm, sidx, didx, rows, acc, sem):
  c = lax.axis_index("c")
  s = lax.axis_index("s")

  zero16 = jnp.zeros((16,), jnp.float32)

  def _zr(i, carry):
    for k in range(DD // 16):
      rows[0, i, pl.ds(k * 16, 16)] = zero16
    return carry
  lax.fori_loop(0, CH, _zr, 0)

  for k in range(STR // CH):
    pltpu.sync_copy(rows.at[0], acc.at[pl.ds(s * STR + k * CH, CH)])
  plsc.subcore_barrier()

  sbase = s * EPT
  dbase = EP + s * EPT
  off = c * NPAD

  def _chunk(j, carry):
    pltpu.sync_copy(e1_hbm.at[pl.ds(sbase + j * CH, CH)], sidx.at[0])
    pltpu.sync_copy(e1_hbm.at[pl.ds(dbase + j * CH, CH)], didx.at[0])

    def _adj(k, c2):
      sidx[0, pl.ds(k * 16, 16)] = sidx[0, pl.ds(k * 16, 16)] + off
      return c2
    lax.fori_loop(0, CH // 16, _adj, 0)

    pltpu.async_copy(z_hbm.at[sidx.at[0]], rows.at[0], sem).wait()
    pltpu.sync_copy(rows.at[0], acc.at[didx.at[0]], add=True)
    return carry
  lax.fori_loop(0, RPT, _chunk, 0)
  plsc.subcore_barrier()

  for k in range(STR // CH):
    pltpu.sync_copy(acc.at[pl.ds(s * STR + k * CH, CH)], rows.at[0])
    pltpu.sync_copy(rows.at[0],
                    out_hbm.at[pl.ds(c * NPAD + s * STR + k * CH, CH)])


def _segsums(z, e1):
  f = functools.partial(
      pl.kernel,
      out_type=jax.ShapeDtypeStruct((2 * NPAD, DD), jnp.float32),
      mesh=_sc_mesh(),
      scratch_types=[
          pltpu.VMEM((2, CH), jnp.int32),
          pltpu.VMEM((2, CH), jnp.int32),
          pltpu.VMEM((2, CH, DD), jnp.float32),
          pltpu.VMEM_SHARED((NPAD, DD), jnp.float32),
          pltpu.SemaphoreType.DMA,
      ],
  )(_feat_body)
  return f(z, e1)


def _elu(v):
  return jnp.where(v > 0, v, jnp.exp(jnp.minimum(v, 0.0)) - 1.0)


def _mix_body(w_ref, x_ref, agg_ref, di_ref, wg_ref, bg_ref, wsl_ref,
              wsr_ref, bs_ref, wgin_ref, bgin_ref, out_ref):
  x = x_ref[...]
  sum_nb = agg_ref[0]
  s_y = agg_ref[1]
  di = di_ref[:, 0]
  agg_gcn = s_y * lax.rsqrt(di)[:, None]
  mean_nb = sum_nb * (1.0 / di)[:, None]
  h_gcn = jnp.dot(agg_gcn, wg_ref[...],
                  preferred_element_type=jnp.float32) + bg_ref[...]
  h_sage = (jnp.dot(x, wsl_ref[...], preferred_element_type=jnp.float32)
            + jnp.dot(mean_nb, wsr_ref[...], preferred_element_type=jnp.float32)
            + bs_ref[...])
  h_gin = jnp.dot(x + sum_nb, wgin_ref[...],
                  preferred_element_type=jnp.float32) + bgin_ref[...]
  out_ref[...] = (w_ref[0] * _elu(h_gcn) + w_ref[1] * _elu(h_sage)
                  + w_ref[2] * _elu(h_gin))


def _mix(weights, x, agg, di_arr, w_gcn, b_gcn, w_sage_l, w_sage_r, b_sage,
         w_gin, b_gin):
  wspec = pl.BlockSpec((DD, DD), lambda i: (0, 0))
  bspec = pl.BlockSpec((1, DD), lambda i: (0, 0))
  return pl.pallas_call(
      _mix_body,
      grid=(NB,),
      in_specs=[
          pl.BlockSpec(memory_space=pltpu.SMEM),
          pl.BlockSpec((BLK, DD), lambda i: (i, 0)),
          pl.BlockSpec((2, BLK, DD), lambda i: (0, i, 0)),
          pl.BlockSpec((BLK, 8), lambda i: (i, 0)),
          wspec, bspec, wspec, wspec, bspec, wspec, bspec,
      ],
      out_specs=pl.BlockSpec((BLK, DD), lambda i: (i, 0)),
      out_shape=jax.ShapeDtypeStruct((NN, DD), jnp.float32),
  )(weights, x, agg, di_arr, w_gcn, b_gcn.reshape(1, DD), w_sage_l, w_sage_r,
    b_sage.reshape(1, DD), w_gin, b_gin.reshape(1, DD))


def kernel(x, x0, weights, edge_index, W_gcn, b_gcn, W_sage_l, W_sage_r,
           b_sage, W_gin, b_gin):
  del x0
  pad = jnp.full((2, EP - EE), NN, dtype=jnp.int32)
  e1 = jnp.concatenate([edge_index, pad], axis=1).reshape(2 * EP)
  partials = _degrees(e1).reshape(2, NS, NR, DD)
  z, di_arr = _planes(x, partials)
  agg = _segsums(z.reshape(2 * NPAD, DD), e1).reshape(2, NPAD, DD)
  return _mix(weights, x, agg, di_arr, W_gcn, b_gcn, W_sage_l, W_sage_r,
              b_sage, W_gin, b_gin)


# speedup vs baseline: 8.7973x; 1.2619x over previous
import functools

import jax
import jax.numpy as jnp
from jax import lax
from jax.experimental import pallas as pl
from jax.experimental.pallas import tpu as pltpu
from jax.experimental.pallas import tpu_sc as plsc

NN = 10000
EE = 320000
DD = 128

NC = 2
NS = 16
CH = 128
EP = 327680
EPT = EP // NS
RPT = EPT // CH
NPAD = 10240
NH = NPAD // 2
NR = NPAD // 16
STR = NPAD // NS
DW = 16

BLK = 1024
NB = NPAD // BLK
HB = NB // 2


def _sc_mesh():
  return plsc.VectorSubcoreMesh(
      core_axis_name="c", subcore_axis_name="s", num_cores=NC, num_subcores=NS)


def _deg_body(e1_hbm, out_hbm, idx_v, acc8, sem):
  c = lax.axis_index("c")
  s = lax.axis_index("s")

  zero16 = jnp.zeros((16,), jnp.float32)

  def _z(i, carry):
    acc8[pl.ds(16 * i, 16)] = zero16
    return carry
  lax.fori_loop(0, NR * DD // 16, _z, 0)

  ones16 = jnp.ones((16,), jnp.float32)
  lanes = lax.iota(jnp.int32, 16)
  lane7 = lanes % 8
  m_lo = lanes < 8
  m_hi = lanes >= 8

  base = (1 - c) * EP + s * EPT

  def _chunk(j, carry):
    pltpu.sync_copy(e1_hbm.at[pl.ds(base + j * CH, CH)], idx_v)

    def _sub(k, c2):
      iv = idx_v[pl.ds(k * 16, 16)]
      flat = jnp.where(iv >= NH, 16 * (iv - NH) + 8, 16 * iv) + lane7
      plsc.addupdate_scatter(acc8, [flat], ones16, mask=m_lo)
      plsc.addupdate_scatter(acc8, [flat], ones16, mask=m_hi)
      return c2
    lax.fori_loop(0, CH // 16, _sub, 0)
    return carry
  lax.fori_loop(0, RPT, _chunk, 0)

  w = (c * NS + s) * NR * DD
  pltpu.sync_copy(acc8, out_hbm.at[pl.ds(w, NR * DD)])


def _degrees(e1):
  f = functools.partial(
      pl.kernel,
      out_type=jax.ShapeDtypeStruct((2 * NS * NR * DD,), jnp.float32),
      mesh=_sc_mesh(),
      compiler_params=pltpu.CompilerParams(needs_layout_passes=False),
      scratch_types=[
          pltpu.VMEM((CH,), jnp.int32),
          pltpu.VMEM((NR * DD,), jnp.float32),
          pltpu.SemaphoreType.DMA,
      ],
  )(_deg_body)
  return f(e1)


def _scale_body(x_ref, part_ref, z_ref, di_ref):
  i = pl.program_id(0)
  xb = x_ref[...]
  t = jnp.sum(part_ref[...], axis=1)
  tt = t.reshape(2, BLK // 8, 8, 16)
  lo = jnp.sum(tt[:, :, :, 0:8], axis=-1).reshape(2, BLK)
  hi = jnp.sum(tt[:, :, :, 8:16], axis=-1).reshape(2, BLK)
  deg = jnp.where(i < HB, lo, hi)
  di = jnp.maximum(deg[0], 1.0)
  do = jnp.maximum(deg[1], 1.0)
  z_ref[0] = xb
  z_ref[1] = xb * lax.rsqrt(do)[:, None]
  di_ref[...] = jnp.broadcast_to(di[:, None], (BLK, 8))


def _planes(x, partials):
  return pl.pallas_call(
      _scale_body,
      grid=(NB,),
      in_specs=[
          pl.BlockSpec((BLK, DD), lambda i: (i, 0)),
          pl.BlockSpec((2, NS, BLK // 8, DD), lambda i: (0, 0, i % HB, 0)),
      ],
      out_specs=[
          pl.BlockSpec((2, BLK, DD), lambda i: (0, i, 0)),
          pl.BlockSpec((BLK, 8), lambda i: (i, 0)),
      ],
      out_shape=[
          jax.ShapeDtypeStruct((2, NPAD, DD), jnp.float32),
          jax.ShapeDtypeStruct((NPAD, 8), jnp.float32),
      ],
  )(x, partials)


def _feat_body(z_hbm, e1_hbm, out_hbm, sidx, didx, rows, acc, sem, sem2):
  c = lax.axis_index("c")
  s = lax.axis_index("s")

  zero16 = jnp.zeros((16,), jnp.float32)

  def _zr(i, carry):
    for k in range(DD // 16):
      rows[0, i, pl.ds(k * 16, 16)] = zero16
    return carry
  lax.fori_loop(0, CH, _zr, 0)

  for k in range(STR // CH):
    pltpu.sync_copy(rows.at[0], acc.at[pl.ds(s * STR + k * CH, CH)])
  plsc.subcore_barrier()

  sbase = s * EPT
  dbase = EP + s * EPT
  off = c * NPAD

  def _load(j, slot):
    pltpu.sync_copy(e1_hbm.at[pl.ds(sbase + j * CH, CH)], sidx.at[slot])
    pltpu.sync_copy(e1_hbm.at[pl.ds(dbase + j * CH, CH)], didx.at[slot])

    def _adj(k, c2):
      sidx[slot, pl.ds(k * 16, 16)] = sidx[slot, pl.ds(k * 16, 16)] + off
      return c2
    lax.fori_loop(0, CH // 16, _adj, 0)

  def _gather(slot, gsem):
    return pltpu.async_copy(z_hbm.at[sidx.at[slot]], rows.at[slot], gsem)

  def _gwait(slot, gsem):
    pltpu.make_async_copy(z_hbm.at[sidx.at[slot]], rows.at[slot], gsem).wait()

  def _scatter(slot):
    pltpu.sync_copy(rows.at[slot], acc.at[didx.at[slot]], add=True)

  _load(0, 0)
  _gather(0, sem)

  def _pair(i, carry):
    j = 2 * i
    _load(j + 1, 1)
    _gwait(0, sem)
    _gather(1, sem2)
    _scatter(0)

    @pl.when(j + 2 < RPT)
    def _():
      _load(j + 2, 0)
    _gwait(1, sem2)

    @pl.when(j + 2 < RPT)
    def _():
      _gather(0, sem)
    _scatter(1)
    return carry
  lax.fori_loop(0, RPT // 2, _pair, 0)
  plsc.subcore_barrier()

  for k in range(STR // CH):
    pltpu.sync_copy(acc.at[pl.ds(s * STR + k * CH, CH)], rows.at[0])
    pltpu.sync_copy(rows.at[0],
                    out_hbm.at[pl.ds(c * NPAD + s * STR + k * CH, CH)])


def _segsums(z, e1):
  f = functools.partial(
      pl.kernel,
      out_type=jax.ShapeDtypeStruct((2 * NPAD, DD), jnp.float32),
      mesh=_sc_mesh(),
      scratch_types=[
          pltpu.VMEM((2, CH), jnp.int32),
          pltpu.VMEM((2, CH), jnp.int32),
          pltpu.VMEM((2, CH, DD), jnp.float32),
          pltpu.VMEM_SHARED((NPAD, DD), jnp.float32),
          pltpu.SemaphoreType.DMA,
          pltpu.SemaphoreType.DMA,
      ],
  )(_feat_body)
  return f(z, e1)


def _elu(v):
  return jnp.where(v > 0, v, jnp.exp(jnp.minimum(v, 0.0)) - 1.0)


def _mix_body(w_ref, x_ref, agg_ref, di_ref, wg_ref, bg_ref, wsl_ref,
              wsr_ref, bs_ref, wgin_ref, bgin_ref, out_ref):
  x = x_ref[...]
  sum_nb = agg_ref[0]
  s_y = agg_ref[1]
  di = di_ref[:, 0]
  agg_gcn = s_y * lax.rsqrt(di)[:, None]
  mean_nb = sum_nb * (1.0 / di)[:, None]
  h_gcn = jnp.dot(agg_gcn, wg_ref[...],
                  preferred_element_type=jnp.float32) + bg_ref[...]
  h_sage = (jnp.dot(x, wsl_ref[...], preferred_element_type=jnp.float32)
            + jnp.dot(mean_nb, wsr_ref[...], preferred_element_type=jnp.float32)
            + bs_ref[...])
  h_gin = jnp.dot(x + sum_nb, wgin_ref[...],
                  preferred_element_type=jnp.float32) + bgin_ref[...]
  out_ref[...] = (w_ref[0] * _elu(h_gcn) + w_ref[1] * _elu(h_sage)
                  + w_ref[2] * _elu(h_gin))


def _mix(weights, x, agg, di_arr, w_gcn, b_gcn, w_sage_l, w_sage_r, b_sage,
         w_gin, b_gin):
  wspec = pl.BlockSpec((DD, DD), lambda i: (0, 0))
  bspec = pl.BlockSpec((1, DD), lambda i: (0, 0))
  return pl.pallas_call(
      _mix_body,
      grid=(NB,),
      in_specs=[
          pl.BlockSpec(memory_space=pltpu.SMEM),
          pl.BlockSpec((BLK, DD), lambda i: (i, 0)),
          pl.BlockSpec((2, BLK, DD), lambda i: (0, i, 0)),
          pl.BlockSpec((BLK, 8), lambda i: (i, 0)),
          wspec, bspec, wspec, wspec, bspec, wspec, bspec,
      ],
      out_specs=pl.BlockSpec((BLK, DD), lambda i: (i, 0)),
      out_shape=jax.ShapeDtypeStruct((NN, DD), jnp.float32),
  )(weights, x, agg, di_arr, w_gcn, b_gcn.reshape(1, DD), w_sage_l, w_sage_r,
    b_sage.reshape(1, DD), w_gin, b_gin.reshape(1, DD))


def kernel(x, x0, weights, edge_index, W_gcn, b_gcn, W_sage_l, W_sage_r,
           b_sage, W_gin, b_gin):
  del x0
  pad = jnp.full((2, EP - EE), NN, dtype=jnp.int32)
  e1 = jnp.concatenate([edge_index, pad], axis=1).reshape(2 * EP)
  partials = _degrees(e1).reshape(2, NS, NR, DD)
  z, di_arr = _planes(x, partials)
  agg = _segsums(z.reshape(2 * NPAD, DD), e1).reshape(2, NPAD, DD)
  return _mix(weights, x, agg, di_arr, W_gcn, b_gcn, W_sage_l, W_sage_r,
              b_sage, W_gin, b_gin)
